# baseline (speedup 1.0000x reference)
import functools

import jax
import jax.numpy as jnp
from jax import lax
from jax.experimental import pallas as pl
from jax.experimental.pallas import tpu as pltpu
from jax.experimental.pallas import tpu_sc as plsc

NC = 2
NS = 16
LANES = 16
WIN = 50
NBUF = 8


def _sc_sum_pool(table, ids2, B, SEQ, D):
    NW = NC * NS
    RPW = B // NW
    WPR = SEQ // WIN
    NWIN = RPW * WPR
    assert NWIN % NBUF == 0 and NBUF % WPR == 0
    mesh = plsc.VectorSubcoreMesh(core_axis_name="c", subcore_axis_name="s")
    CG = D // LANES

    @functools.partial(
        pl.kernel,
        out_type=jax.ShapeDtypeStruct((B, D), jnp.float32),
        mesh=mesh,
        scratch_types=[
            pltpu.VMEM((NWIN, WIN), jnp.int32),
        ] + [pltpu.VMEM((WIN, D), jnp.float32) for _ in range(NBUF)] + [
            pltpu.VMEM((2, D), jnp.float32),
        ] + [pltpu.SemaphoreType.DMA for _ in range(NBUF + 1)],
    )
    def sc_kernel(table_hbm, ids_hbm, out_hbm, ids_v, *rest):
        bufs = rest[:NBUF]
        out_stage = rest[NBUF]
        sems = rest[NBUF + 1:NBUF + 1 + NBUF]
        osem = rest[NBUF + 1 + NBUF]
        wid = lax.axis_index("s") * NC + lax.axis_index("c")
        base = wid * NWIN
        pltpu.sync_copy(ids_hbm.at[pl.ds(base, NWIN)], ids_v)

        def issue(b, w):
            pltpu.async_copy(table_hbm.at[ids_v.at[w]], bufs[b], sems[b])

        def wait(b, w):
            pltpu.make_async_copy(table_hbm.at[ids_v.at[w]], bufs[b], sems[b]).wait()

        def accumulate(buf, accs):
            @plsc.parallel_loop(0, WIN, unroll=2, carry=accs)
            def body(j, a):
                return tuple(
                    a[c] + buf[j, pl.ds(c * LANES, LANES)] for c in range(CG)
                )

            return body

        for b in range(NBUF - 1):
            issue(b, b)

        zeros = (jnp.zeros((LANES,), jnp.float32),) * CG

        @pl.loop(0, NWIN, step=NBUF)
        def _(w0):
            accs = zeros
            for b in range(NBUF):
                w = w0 + b

                @pl.when(w + NBUF - 1 < NWIN)
                def _():
                    issue((b + NBUF - 1) % NBUF, w + NBUF - 1)

                wait(b, w)
                accs = accumulate(bufs[b], accs)
                if b % WPR == WPR - 1:
                    r = (w0 + b) // WPR
                    p = ((b + 1) // WPR - 1) % 2

                    @pl.when(r >= 2)
                    def _():
                        pltpu.make_async_copy(
                            out_stage.at[pl.ds(p, 1)],
                            out_hbm.at[pl.ds(wid * RPW + r - 2, 1)],
                            osem,
                        ).wait()

                    for c in range(CG):
                        out_stage[p, pl.ds(c * LANES, LANES)] = accs[c]
                    pltpu.async_copy(
                        out_stage.at[pl.ds(p, 1)],
                        out_hbm.at[pl.ds(wid * RPW + r, 1)],
                        osem,
                    )
                    accs = zeros

        for p, r in ((0, RPW - 2), (1, RPW - 1)):
            pltpu.make_async_copy(
                out_stage.at[pl.ds(p, 1)],
                out_hbm.at[pl.ds(wid * RPW + r, 1)],
                osem,
            ).wait()

    return sc_kernel(table, ids2)


def _tc_counts(ids, B):
    def body(ids_ref, out_ref):
        idv = ids_ref[...]
        out_ref[...] = jnp.sum((idv != 0).astype(jnp.float32), axis=1,
                               keepdims=True)

    return pl.pallas_call(
        body,
        out_shape=jax.ShapeDtypeStruct((B, 1), jnp.float32),
    )(ids)


def _tc_finish(cnt, sums, Wt, b2, B, SEQ, D):
    def tc_body(cnt_ref, sums_ref, wt_ref, b_ref, out_ref):
        cnt = cnt_ref[...]
        pooled = sums_ref[...] / jnp.maximum(cnt, 1.0)
        h = jnp.maximum(pooled, 0.0)
        h = lax.dot_general(h, wt_ref[...], (((1,), (1,)), ((), ())),
                            preferred_element_type=jnp.float32)
        h = h + b_ref[...]
        nrm = jnp.maximum(jnp.sqrt(jnp.sum(h * h, axis=1, keepdims=True)), 1e-12)
        out_ref[...] = h / nrm

    return pl.pallas_call(
        tc_body,
        out_shape=jax.ShapeDtypeStruct((B, D), jnp.float32),
    )(cnt, sums, Wt, b2)


def kernel(ids, table, W, b):
    B, SEQ = ids.shape
    V, D = table.shape
    CH = 1
    Bc = B // CH
    Wt = W
    b2 = b.reshape(1, D)
    zs = []
    for i in range(CH):
        idc = lax.slice(ids, (i * Bc, 0), ((i + 1) * Bc, SEQ))
        ids2 = idc.reshape(Bc * (SEQ // WIN), WIN)
        sums = _sc_sum_pool(table, ids2, Bc, SEQ, D)
        cnt = _tc_counts(idc, Bc)
        zs.append(_tc_finish(cnt, sums, Wt, b2, Bc, SEQ, D))
    return jnp.concatenate(zs, axis=0)


# speedup vs baseline: 1.0052x; 1.0052x over previous
import functools

import jax
import jax.numpy as jnp
from jax import lax
from jax.experimental import pallas as pl
from jax.experimental.pallas import tpu as pltpu
from jax.experimental.pallas import tpu_sc as plsc

NC = 2
NS = 16
LANES = 16
WIN = 50
NBUF = 8


def _sc_sum_pool(table, ids2, B, SEQ, D):
    NW = NC * NS
    RPW = B // NW
    WPR = SEQ // WIN
    NWIN = RPW * WPR
    assert NWIN % NBUF == 0 and NBUF % WPR == 0
    mesh = plsc.VectorSubcoreMesh(core_axis_name="c", subcore_axis_name="s")
    CG = D // LANES

    @functools.partial(
        pl.kernel,
        out_type=jax.ShapeDtypeStruct((B, D), jnp.float32),
        mesh=mesh,
        scratch_types=[
            pltpu.VMEM((NWIN, WIN), jnp.int32),
        ] + [pltpu.VMEM((WIN, D), jnp.float32) for _ in range(NBUF)] + [
            pltpu.VMEM((2, D), jnp.float32),
        ] + [pltpu.SemaphoreType.DMA for _ in range(NBUF + 1)],
    )
    def sc_kernel(table_hbm, ids_hbm, out_hbm, ids_v, *rest):
        bufs = rest[:NBUF]
        out_stage = rest[NBUF]
        sems = rest[NBUF + 1:NBUF + 1 + NBUF]
        osem = rest[NBUF + 1 + NBUF]
        wid = lax.axis_index("s") * NC + lax.axis_index("c")
        base = wid * NWIN
        pltpu.sync_copy(ids_hbm.at[pl.ds(base, NWIN)], ids_v)

        def issue(b, w):
            pltpu.async_copy(table_hbm.at[ids_v.at[w]], bufs[b], sems[b])

        def wait(b, w):
            pltpu.make_async_copy(table_hbm.at[ids_v.at[w]], bufs[b], sems[b]).wait()

        def accumulate(buf, accs):
            @plsc.parallel_loop(0, WIN, unroll=2, carry=accs)
            def body(j, a):
                return tuple(
                    a[c] + buf[j, pl.ds(c * LANES, LANES)] for c in range(CG)
                )

            return body

        for b in range(NBUF - 1):
            issue(b, b)

        zeros = (jnp.zeros((LANES,), jnp.float32),) * CG

        @pl.loop(0, NWIN, step=NBUF)
        def _(w0):
            accs = zeros
            for b in range(NBUF):
                w = w0 + b

                @pl.when(w + NBUF - 1 < NWIN)
                def _():
                    issue((b + NBUF - 1) % NBUF, w + NBUF - 1)

                wait(b, w)
                accs = accumulate(bufs[b], accs)
                if b % WPR == WPR - 1:
                    r = (w0 + b) // WPR
                    p = ((b + 1) // WPR - 1) % 2

                    @pl.when(r >= 2)
                    def _():
                        pltpu.make_async_copy(
                            out_stage.at[pl.ds(p, 1)],
                            out_hbm.at[pl.ds(wid * RPW + r - 2, 1)],
                            osem,
                        ).wait()

                    for c in range(CG):
                        out_stage[p, pl.ds(c * LANES, LANES)] = accs[c]
                    pltpu.async_copy(
                        out_stage.at[pl.ds(p, 1)],
                        out_hbm.at[pl.ds(wid * RPW + r, 1)],
                        osem,
                    )
                    accs = zeros

        for p, r in ((0, RPW - 2), (1, RPW - 1)):
            pltpu.make_async_copy(
                out_stage.at[pl.ds(p, 1)],
                out_hbm.at[pl.ds(wid * RPW + r, 1)],
                osem,
            ).wait()

    return sc_kernel(table, ids2)


def _tc_finish(ids, sums, Wt, b2, B, SEQ, D):
    def tc_body(ids_ref, sums_ref, wt_ref, b_ref, out_ref):
        idv = ids_ref[...]
        cnt = jnp.sum((idv != 0).astype(jnp.float32), axis=1, keepdims=True)
        pooled = sums_ref[...] / jnp.maximum(cnt, 1.0)
        h = jnp.maximum(pooled, 0.0)
        h = lax.dot_general(h, wt_ref[...], (((1,), (1,)), ((), ())),
                            preferred_element_type=jnp.float32)
        h = h + b_ref[...]
        nrm = jnp.maximum(jnp.sqrt(jnp.sum(h * h, axis=1, keepdims=True)), 1e-12)
        out_ref[...] = h / nrm

    return pl.pallas_call(
        tc_body,
        out_shape=jax.ShapeDtypeStruct((B, D), jnp.float32),
    )(ids, sums, Wt, b2)


def kernel(ids, table, W, b):
    B, SEQ = ids.shape
    V, D = table.shape
    CH = 1
    Bc = B // CH
    Wt = W
    b2 = b.reshape(1, D)
    zs = []
    for i in range(CH):
        idc = lax.slice(ids, (i * Bc, 0), ((i + 1) * Bc, SEQ))
        ids2 = idc.reshape(Bc * (SEQ // WIN), WIN)
        sums = _sc_sum_pool(table, ids2, Bc, SEQ, D)
        zs.append(_tc_finish(idc, sums, Wt, b2, Bc, SEQ, D))
    return jnp.concatenate(zs, axis=0)


# speedup vs baseline: 1.0081x; 1.0029x over previous
import functools

import jax
import jax.numpy as jnp
from jax import lax
from jax.experimental import pallas as pl
from jax.experimental.pallas import tpu as pltpu
from jax.experimental.pallas import tpu_sc as plsc

NC = 2
NS = 16
LANES = 16
WIN = 50
NBUF = 8


def _sc_sum_pool(table, ids2, B, SEQ, D):
    NW = NC * NS
    RPW = B // NW
    WPR = SEQ // WIN
    NWIN = RPW * WPR
    assert NWIN % NBUF == 0 and NBUF % WPR == 0
    mesh = plsc.VectorSubcoreMesh(core_axis_name="c", subcore_axis_name="s")
    CG = D // LANES

    @functools.partial(
        pl.kernel,
        out_type=jax.ShapeDtypeStruct((B, D), jnp.float32),
        mesh=mesh,
        scratch_types=[
            pltpu.VMEM((NWIN, WIN), jnp.int32),
        ] + [pltpu.VMEM((WIN, D), jnp.float32) for _ in range(NBUF)] + [
            pltpu.VMEM((2, D), jnp.float32),
        ] + [pltpu.SemaphoreType.DMA for _ in range(NBUF + 2)],
    )
    def sc_kernel(table_hbm, ids_hbm, out_hbm, ids_v, *rest):
        bufs = rest[:NBUF]
        out_stage = rest[NBUF]
        sems = rest[NBUF + 1:NBUF + 1 + NBUF]
        osems = rest[NBUF + 1 + NBUF:]
        wid = lax.axis_index("s") * NC + lax.axis_index("c")
        base = wid * NWIN
        pltpu.sync_copy(ids_hbm.at[pl.ds(base, NWIN)], ids_v)

        def issue(b, w):
            pltpu.async_copy(table_hbm.at[ids_v.at[w]], bufs[b], sems[b])

        def wait(b, w):
            pltpu.make_async_copy(table_hbm.at[ids_v.at[w]], bufs[b], sems[b]).wait()

        def accumulate(buf, accs):
            @plsc.parallel_loop(0, WIN, unroll=2, carry=accs)
            def body(j, a):
                return tuple(
                    a[c] + buf[j, pl.ds(c * LANES, LANES)] for c in range(CG)
                )

            return body

        for b in range(NBUF - 1):
            issue(b, b)

        zeros = (jnp.zeros((LANES,), jnp.float32),) * CG

        @pl.loop(0, NWIN, step=NBUF)
        def _(w0):
            accs = zeros
            for b in range(NBUF):
                w = w0 + b

                @pl.when(w + NBUF - 1 < NWIN)
                def _():
                    issue((b + NBUF - 1) % NBUF, w + NBUF - 1)

                wait(b, w)
                accs = accumulate(bufs[b], accs)
                if b % WPR == WPR - 1:
                    r = (w0 + b) // WPR
                    p = ((b + 1) // WPR - 1) % 2

                    @pl.when(r >= 2)
                    def _():
                        pltpu.make_async_copy(
                            out_stage.at[pl.ds(p, 1)],
                            out_hbm.at[pl.ds(wid * RPW + r - 2, 1)],
                            osems[p],
                        ).wait()

                    for c in range(CG):
                        out_stage[p, pl.ds(c * LANES, LANES)] = accs[c]
                    pltpu.async_copy(
                        out_stage.at[pl.ds(p, 1)],
                        out_hbm.at[pl.ds(wid * RPW + r, 1)],
                        osems[p],
                    )
                    accs = zeros

        for p, r in ((0, RPW - 2), (1, RPW - 1)):
            pltpu.make_async_copy(
                out_stage.at[pl.ds(p, 1)],
                out_hbm.at[pl.ds(wid * RPW + r, 1)],
                osems[p],
            ).wait()

    return sc_kernel(table, ids2)


def _tc_finish(ids, sums, Wt, b2, B, SEQ, D):
    def tc_body(ids_ref, sums_ref, wt_ref, b_ref, out_ref):
        idv = ids_ref[...]
        cnt = jnp.sum((idv != 0).astype(jnp.float32), axis=1, keepdims=True)
        pooled = sums_ref[...] / jnp.maximum(cnt, 1.0)
        h = jnp.maximum(pooled, 0.0)
        h = lax.dot_general(h, wt_ref[...], (((1,), (1,)), ((), ())),
                            preferred_element_type=jnp.float32)
        h = h + b_ref[...]
        nrm = jnp.maximum(jnp.sqrt(jnp.sum(h * h, axis=1, keepdims=True)), 1e-12)
        out_ref[...] = h / nrm

    return pl.pallas_call(
        tc_body,
        out_shape=jax.ShapeDtypeStruct((B, D), jnp.float32),
    )(ids, sums, Wt, b2)


def kernel(ids, table, W, b):
    B, SEQ = ids.shape
    V, D = table.shape
    CH = 1
    Bc = B // CH
    Wt = W
    b2 = b.reshape(1, D)
    zs = []
    for i in range(CH):
        idc = lax.slice(ids, (i * Bc, 0), ((i + 1) * Bc, SEQ))
        ids2 = idc.reshape(Bc * (SEQ // WIN), WIN)
        sums = _sc_sum_pool(table, ids2, Bc, SEQ, D)
        zs.append(_tc_finish(idc, sums, Wt, b2, Bc, SEQ, D))
    return jnp.concatenate(zs, axis=0)
